# baseline (speedup 1.0000x reference)
import jax
import jax.numpy as jnp
from jax.experimental import pallas as pl
from jax.experimental.pallas import tpu as pltpu

G, S, DIM, E, TOPN = 10, 2048, 2560, 8, 2
INTER = 6912
C = 384
M = G * C
MC = 768
NM = M // MC
HB = 384
NH = INTER // HB
NSLOT = E * M


def _logits_body(x_ref, w_ref, o_ref):
    o_ref[...] = jnp.dot(x_ref[...].astype(jnp.float32), w_ref[...],
                         preferred_element_type=jnp.float32)


def _router_logits(xf, router_gate):
    wpad = jnp.zeros((DIM, 128), jnp.float32).at[:, :E].set(router_gate)
    out = pl.pallas_call(
        _logits_body,
        grid=(G,),
        in_specs=[pl.BlockSpec((S, DIM), lambda i: (i, 0)),
                  pl.BlockSpec((DIM, 128), lambda i: (0, 0))],
        out_specs=pl.BlockSpec((S, 128), lambda i: (i, 0)),
        out_shape=jax.ShapeDtypeStruct((G * S, 128), jnp.float32),
    )(xf, wpad)
    return out[:, :E]


def _mlp_body(x_ref, wg_ref, wi_ref, wo_ref, o_ref, acc_ref):
    h = pl.program_id(2)
    x = x_ref[0]
    h1 = jnp.dot(x, wg_ref[0], preferred_element_type=jnp.float32)
    h0 = jnp.dot(x, wi_ref[0], preferred_element_type=jnp.float32)
    hh = jax.nn.silu(h1.astype(jnp.bfloat16)) * h0.astype(jnp.bfloat16)
    y = jnp.dot(hh, wo_ref[0], preferred_element_type=jnp.float32)

    @pl.when(h == 0)
    def _():
        acc_ref[...] = y

    @pl.when(h > 0)
    def _():
        acc_ref[...] += y

    @pl.when(h == NH - 1)
    def _():
        o_ref[0] = acc_ref[...].astype(jnp.bfloat16)


def _expert_mlp(Xe, wg, wi, wo):
    return pl.pallas_call(
        _mlp_body,
        grid=(E, NM, NH),
        in_specs=[
            pl.BlockSpec((1, MC, DIM), lambda e, m, h: (e, m, 0)),
            pl.BlockSpec((1, DIM, HB), lambda e, m, h: (e, 0, h)),
            pl.BlockSpec((1, DIM, HB), lambda e, m, h: (e, 0, h)),
            pl.BlockSpec((1, HB, DIM), lambda e, m, h: (e, h, 0)),
        ],
        out_specs=pl.BlockSpec((1, MC, DIM), lambda e, m, h: (e, m, 0)),
        out_shape=jax.ShapeDtypeStruct((E, M, DIM), jnp.bfloat16),
        scratch_shapes=[pltpu.VMEM((MC, DIM), jnp.float32)],
        compiler_params=pltpu.CompilerParams(
            dimension_semantics=("arbitrary", "arbitrary", "arbitrary")),
    )(Xe, wg, wi, wo)


def _route(logits):
    max1 = jnp.max(logits, axis=-1)
    e_ids = jnp.arange(E, dtype=jnp.int32)
    arg1 = jnp.min(jnp.where(logits == max1[..., None], e_ids, E), axis=-1)
    masked = jnp.where(e_ids == arg1[..., None], -jnp.inf, logits)
    max2 = jnp.max(masked, axis=-1)
    arg2 = jnp.min(jnp.where(masked == max2[..., None], e_ids, E), axis=-1)
    b = jnp.exp(max2 - max1)
    g1 = 1.0 / (1.0 + b)
    g2 = b / (1.0 + b)

    ex = jnp.stack([arg1, arg2], axis=-1).reshape(G, S * TOPN)
    gt = jnp.stack([g1, g2], axis=-1).reshape(G, S * TOPN)
    oh = jax.nn.one_hot(ex, E, dtype=jnp.int32)
    pos = jnp.sum((jnp.cumsum(oh, axis=1) - oh) * oh, axis=-1)
    ok = pos < C
    gt = jnp.where(ok, gt, 0.0)

    slot = (ex * G + jnp.arange(G, dtype=jnp.int32)[:, None]) * C + pos
    tok_id = (jnp.arange(G, dtype=jnp.int32)[:, None] * S
              + (jnp.arange(S * TOPN, dtype=jnp.int32) // TOPN)[None, :])
    slot_token = jnp.zeros((NSLOT,), jnp.int32).at[
        jnp.where(ok, slot, NSLOT).ravel()].set(tok_id.ravel(), mode='drop')
    cidx = jnp.where(ok, slot, 0).reshape(G, S, TOPN)
    return slot_token, cidx, gt.reshape(G, S, TOPN)


def kernel(inputs, wi_gate_0, wi_0, wo_0, router_gate):
    bf = inputs.dtype
    xf = inputs.reshape(G * S, DIM)
    logits = _router_logits(xf, router_gate).reshape(G, S, E)
    slot_token, cidx, gt = _route(logits)

    Xe = xf[slot_token].reshape(E, M, DIM)
    Y = _expert_mlp(Xe, wi_gate_0.astype(bf), wi_0.astype(bf), wo_0.astype(bf))
    Yf = Y.reshape(NSLOT, DIM)

    out = (Yf[cidx[..., 0]].astype(jnp.float32) * gt[..., 0:1]
           + Yf[cidx[..., 1]].astype(jnp.float32) * gt[..., 1:2]).astype(bf)
    return out.reshape(inputs.shape)
